# Initial kernel scaffold; baseline (speedup 1.0000x reference)
import jax
import jax.numpy as jnp
from jax.experimental import pallas as pl


def kernel(direct_effects, edge_index, W):
    raise NotImplementedError("write your pallas kernel here")


# speedup vs baseline: 3.6039x; 3.6039x over previous
import functools

import jax
import jax.numpy as jnp
from jax import lax
from jax.experimental import pallas as pl
from jax.experimental.pallas import tpu as pltpu
from jax.experimental.pallas import tpu_sc as plsc

N_GENES = 50000
N_EDGES = 1600000
BATCH = 32
K_STEPS = 3
LANES = 16
NUM_CORES = 2

CHUNK = 4000
NUM_CHUNKS = N_EDGES // CHUNK
GROUPS_PER_CHUNK = CHUNK // LANES
X_GROUPS = N_GENES // LANES


def _body(de_hbm, src_hbm, dst_hbm, w_hbm, out_hbm, x_v, y_v, src_v, dst_v, w_v):
    wid = lax.axis_index("s") * NUM_CORES + lax.axis_index("c")
    pltpu.sync_copy(de_hbm.at[wid], x_v)

    for _ in range(K_STEPS):
        @pl.loop(0, X_GROUPS)
        def _zero(i):
            y_v[pl.ds(i * LANES, LANES)] = jnp.zeros((LANES,), jnp.float32)

        @pl.loop(0, NUM_CHUNKS)
        def _chunk(c):
            base = c * CHUNK
            pltpu.sync_copy(src_hbm.at[pl.ds(base, CHUNK)], src_v)
            pltpu.sync_copy(dst_hbm.at[pl.ds(base, CHUNK)], dst_v)
            pltpu.sync_copy(w_hbm.at[pl.ds(base, CHUNK)], w_v)

            @pl.loop(0, GROUPS_PER_CHUNK)
            def _group(g):
                off = g * LANES
                s = src_v[pl.ds(off, LANES)]
                d = dst_v[pl.ds(off, LANES)]
                w = w_v[pl.ds(off, LANES)]
                xv = plsc.load_gather(x_v, [s])
                plsc.addupdate_scatter(y_v, [d], xv * w)

        @pl.loop(0, X_GROUPS)
        def _acc(i):
            sl = pl.ds(i * LANES, LANES)
            x_v[sl] = x_v[sl] + y_v[sl]

    pltpu.sync_copy(x_v, out_hbm.at[wid])


@jax.jit
def _run(direct_effects, edge_src, edge_dst, w):
    mesh = plsc.VectorSubcoreMesh(core_axis_name="c", subcore_axis_name="s")
    return pl.kernel(
        _body,
        out_type=jax.ShapeDtypeStruct((BATCH, N_GENES), jnp.float32),
        mesh=mesh,
        scratch_types=[
            pltpu.VMEM((N_GENES,), jnp.float32),
            pltpu.VMEM((N_GENES,), jnp.float32),
            pltpu.VMEM((CHUNK,), jnp.int32),
            pltpu.VMEM((CHUNK,), jnp.int32),
            pltpu.VMEM((CHUNK,), jnp.float32),
        ],
        compiler_params=pltpu.CompilerParams(needs_layout_passes=False),
    )(direct_effects, edge_src, edge_dst, w)


def kernel(direct_effects, edge_index, W):
    edge_src = edge_index[0].astype(jnp.int32)
    edge_dst = edge_index[1].astype(jnp.int32)
    return _run(direct_effects, edge_src, edge_dst, W)


# speedup vs baseline: 15.5820x; 4.3237x over previous
import functools

import jax
import jax.numpy as jnp
from jax import lax
from jax.experimental import pallas as pl
from jax.experimental.pallas import tpu as pltpu
from jax.experimental.pallas import tpu_sc as plsc

N_GENES = 50000
N_EDGES = 1600000
BATCH = 32
K_STEPS = 3
LANES = 16
NUM_CORES = 2

CHUNK = 4000
NUM_CHUNKS = N_EDGES // CHUNK
GROUPS_PER_CHUNK = CHUNK // LANES
X_GROUPS = N_GENES // LANES
NBUF = 2


def _body(de_hbm, src_hbm, dst_hbm, w_hbm, out_hbm,
          x_v, y_v, src0_v, src1_v, dst0_v, dst1_v, w0_v, w1_v, sem0, sem1):
    sems = (sem0, sem1)
    srcs = (src0_v, src1_v)
    dsts = (dst0_v, dst1_v)
    ws = (w0_v, w1_v)
    wid = lax.axis_index("s") * NUM_CORES + lax.axis_index("c")
    pltpu.sync_copy(de_hbm.at[wid], x_v)

    def issue(c, b):
        base = c * CHUNK
        pltpu.async_copy(src_hbm.at[pl.ds(base, CHUNK)], srcs[b], sems[b])
        pltpu.async_copy(dst_hbm.at[pl.ds(base, CHUNK)], dsts[b], sems[b])
        pltpu.async_copy(w_hbm.at[pl.ds(base, CHUNK)], ws[b], sems[b])

    def drain(b):
        pltpu.make_async_copy(src_hbm.at[pl.ds(0, CHUNK)], srcs[b], sems[b]).wait()
        pltpu.make_async_copy(dst_hbm.at[pl.ds(0, CHUNK)], dsts[b], sems[b]).wait()
        pltpu.make_async_copy(w_hbm.at[pl.ds(0, CHUNK)], ws[b], sems[b]).wait()

    for _ in range(K_STEPS):
        @pl.loop(0, X_GROUPS, unroll=8)
        def _zero(i):
            y_v[pl.ds(i * LANES, LANES)] = jnp.zeros((LANES,), jnp.float32)

        for b in range(NBUF):
            issue(b, b)

        @pl.loop(0, NUM_CHUNKS, step=NBUF)
        def _chunk(c0):
            for b in range(NBUF):
                drain(b)

                @plsc.parallel_loop(0, GROUPS_PER_CHUNK, unroll=8)
                def _group(g):
                    off = g * LANES
                    s = srcs[b][pl.ds(off, LANES)]
                    d = dsts[b][pl.ds(off, LANES)]
                    w = ws[b][pl.ds(off, LANES)]
                    xv = plsc.load_gather(x_v, [s])
                    plsc.addupdate_scatter(y_v, [d], xv * w)

                nxt = c0 + b + NBUF

                @pl.when(nxt < NUM_CHUNKS)
                def _():
                    issue(nxt, b)

        @pl.loop(0, X_GROUPS, unroll=8)
        def _acc(i):
            sl = pl.ds(i * LANES, LANES)
            x_v[sl] = x_v[sl] + y_v[sl]

    pltpu.sync_copy(x_v, out_hbm.at[wid])


@jax.jit
def _run(direct_effects, edge_src, edge_dst, w):
    mesh = plsc.VectorSubcoreMesh(core_axis_name="c", subcore_axis_name="s")
    return pl.kernel(
        _body,
        out_type=jax.ShapeDtypeStruct((BATCH, N_GENES), jnp.float32),
        mesh=mesh,
        scratch_types=[
            pltpu.VMEM((N_GENES,), jnp.float32),
            pltpu.VMEM((N_GENES,), jnp.float32),
            pltpu.VMEM((CHUNK,), jnp.int32),
            pltpu.VMEM((CHUNK,), jnp.int32),
            pltpu.VMEM((CHUNK,), jnp.int32),
            pltpu.VMEM((CHUNK,), jnp.int32),
            pltpu.VMEM((CHUNK,), jnp.float32),
            pltpu.VMEM((CHUNK,), jnp.float32),
            pltpu.SemaphoreType.DMA,
            pltpu.SemaphoreType.DMA,
        ],
        compiler_params=pltpu.CompilerParams(needs_layout_passes=False),
    )(direct_effects, edge_src, edge_dst, w)


def kernel(direct_effects, edge_index, W):
    edge_src = edge_index[0].astype(jnp.int32)
    edge_dst = edge_index[1].astype(jnp.int32)
    return _run(direct_effects, edge_src, edge_dst, W)
